# baseline (speedup 1.0000x reference)
import jax
import jax.numpy as jnp
from jax import lax
from jax.experimental import pallas as pl
from jax.experimental.pallas import tpu as pltpu
from jax.experimental.pallas import tpu_sc as plsc

VOCAB = 1000000
EMB = 64
B = 4096
L = 200

NC = 2
NS = 16
NW = NC * NS
LANES = 16

TOTAL = B * L
BPW = TOTAL // NW
G = 128
NGRP = BPW // G
K = 4
C = K * G
NCHUNK = NGRP // K


def _emb_kernel(table, idx3, out, mask3, idx_v, mask_v, rows0, rows1,
                sg0, sg1, so, sm):
    cid = lax.axis_index("c")
    sid = lax.axis_index("s")
    wid = sid * NC + cid
    base = wid * BPW

    rows = (rows0, rows1)
    sg = (sg0, sg1)

    pltpu.sync_copy(idx3.at[wid], idx_v)

    def fire_gathers(c, b):
        for j in range(K):
            pltpu.async_copy(
                table.at[idx_v.at[c * K + j]],
                rows[b].at[pl.ds(j * G, G)],
                sg[b],
            )

    def wait_gathers(c, b):
        for j in range(K):
            pltpu.make_async_copy(
                table.at[idx_v.at[c * K + j]],
                rows[b].at[pl.ds(j * G, G)],
                sg[b],
            ).wait()

    def compute_mask(c):
        one = jnp.full((LANES,), 1, dtype=jnp.int32)
        zero = jnp.full((LANES,), 0, dtype=jnp.int32)
        for j in range(K):
            g = c * K + j
            for v in range(G // LANES):
                sl = pl.ds(v * LANES, LANES)
                vec = idx_v[g, sl]
                mask_v[g, sl] = jnp.where(vec != 0, one, zero)

    fire_gathers(0, 0)
    fire_gathers(1, 1)

    def body(i, carry):
        c0 = i * 2
        for b in range(2):
            c = c0 + b
            wait_gathers(c, b)
            out_cp = pltpu.make_async_copy(
                rows[b],
                out.at[pl.ds(base + c * C, C)],
                so,
            )
            out_cp.start()
            compute_mask(c)
            out_cp.wait()

            @pl.when(c + 2 < NCHUNK)
            def _():
                fire_gathers(c + 2, b)

        return carry

    lax.fori_loop(0, NCHUNK // 2, body, 0)

    mask_cp = pltpu.make_async_copy(mask_v, mask3.at[wid], sm)
    mask_cp.start()
    mask_cp.wait()


@jax.jit
def _run(idx3, embeddings):
    kcall = pl.kernel(
        _emb_kernel,
        out_type=(
            jax.ShapeDtypeStruct((TOTAL, EMB), jnp.float32),
            jax.ShapeDtypeStruct((NW, NGRP, G), jnp.int32),
        ),
        mesh=plsc.VectorSubcoreMesh(core_axis_name="c", subcore_axis_name="s"),
        compiler_params=pltpu.CompilerParams(use_tc_tiling_on_sc=False),
        scratch_types=[
            pltpu.VMEM((NGRP, G), jnp.int32),
            pltpu.VMEM((NGRP, G), jnp.int32),
            pltpu.VMEM((C, EMB), jnp.float32),
            pltpu.VMEM((C, EMB), jnp.float32),
            pltpu.SemaphoreType.DMA,
            pltpu.SemaphoreType.DMA,
            pltpu.SemaphoreType.DMA,
            pltpu.SemaphoreType.DMA,
        ],
    )
    return kcall(embeddings, idx3)


def kernel(encoded, embeddings):
    idx3 = encoded.reshape(NW, NGRP, G)
    out, mask3 = _run(idx3, embeddings)
    emb = out.reshape(B, L, EMB)
    mask = mask3.reshape(B, L)
    return emb, mask
